# Initial kernel scaffold; baseline (speedup 1.0000x reference)
import jax
import jax.numpy as jnp
from jax.experimental import pallas as pl


def kernel(node_emb, edge_index, edge_attr, W1, b1, W2, b2):
    raise NotImplementedError("write your pallas kernel here")


# speedup vs baseline: 2.9311x; 2.9311x over previous
import functools

import jax
import jax.numpy as jnp
from jax import lax
from jax.experimental import pallas as pl
from jax.experimental.pallas import tpu as pltpu
from jax.experimental.pallas import tpu_sc as plsc

N_NODES = 10000
N_EDGES = 320000
HIDDEN = 128
EDGE_DIM = 16

NC = 2
NS = 16
NW = NC * NS
EPW = N_EDGES // NW
CHUNK = 80
NCHUNK = EPW // CHUNK

E_BLK = 2560
N_EBLK = N_EDGES // E_BLK

NODE_BLK = 1000
N_NBLK = N_NODES // NODE_BLK


def _ab_body(ne_ref, wa_ref, wb_ref, a_ref, b_ref):
    x = ne_ref[...]
    a_ref[...] = jnp.dot(x, wa_ref[...], preferred_element_type=jnp.float32)
    b_ref[...] = jnp.dot(x, wb_ref[...], preferred_element_type=jnp.float32)


def _precompute_ab(ne, wa, wb):
    return pl.pallas_call(
        _ab_body,
        grid=(N_NBLK,),
        in_specs=[
            pl.BlockSpec((NODE_BLK, HIDDEN), lambda i: (i, 0)),
            pl.BlockSpec((HIDDEN, HIDDEN), lambda i: (0, 0)),
            pl.BlockSpec((HIDDEN, HIDDEN), lambda i: (0, 0)),
        ],
        out_specs=[
            pl.BlockSpec((NODE_BLK, HIDDEN), lambda i: (i, 0)),
            pl.BlockSpec((NODE_BLK, HIDDEN), lambda i: (i, 0)),
        ],
        out_shape=[
            jax.ShapeDtypeStruct((N_NODES, HIDDEN), jnp.float32),
            jax.ShapeDtypeStruct((N_NODES, HIDDEN), jnp.float32),
        ],
    )(ne, wa, wb)


def _gather_body(a_hbm, b_hbm, src_hbm, dst_hbm, ga_hbm, gb_hbm,
                 src_v, dst_v, rows_a, rows_b, sem_a, sem_b):
    c = lax.axis_index("c")
    s = lax.axis_index("s")
    wid = s * NC + c
    pltpu.sync_copy(src_hbm.at[wid], src_v)
    pltpu.sync_copy(dst_hbm.at[wid], dst_v)

    def body(j, carry):
        cpa = pltpu.async_copy(a_hbm.at[src_v.at[j]], rows_a, sem_a)
        cpb = pltpu.async_copy(b_hbm.at[dst_v.at[j]], rows_b, sem_b)
        cpa.wait()
        cpb.wait()
        base = wid * EPW + j * CHUNK
        pltpu.sync_copy(rows_a, ga_hbm.at[pl.ds(base, CHUNK)])
        pltpu.sync_copy(rows_b, gb_hbm.at[pl.ds(base, CHUNK)])
        return carry

    lax.fori_loop(0, NCHUNK, body, 0)


def _gather_edges(a, b, src, dst):
    mesh = plsc.VectorSubcoreMesh(core_axis_name="c", subcore_axis_name="s")
    fn = pl.kernel(
        _gather_body,
        out_type=[
            jax.ShapeDtypeStruct((N_EDGES, HIDDEN), jnp.float32),
            jax.ShapeDtypeStruct((N_EDGES, HIDDEN), jnp.float32),
        ],
        mesh=mesh,
        scratch_types=[
            pltpu.VMEM((NCHUNK, CHUNK), jnp.int32),
            pltpu.VMEM((NCHUNK, CHUNK), jnp.int32),
            pltpu.VMEM((CHUNK, HIDDEN), jnp.float32),
            pltpu.VMEM((CHUNK, HIDDEN), jnp.float32),
            pltpu.SemaphoreType.DMA,
            pltpu.SemaphoreType.DMA,
        ],
    )
    return fn(a, b, src, dst)


def _mlp_body(ga_ref, gb_ref, ea_ref, wc_ref, b1_ref, w2_ref, b2_ref, out_ref):
    e = jnp.dot(ea_ref[...], wc_ref[...], preferred_element_type=jnp.float32)
    h = ga_ref[...] + gb_ref[...] + e + b1_ref[...]
    h = jnp.maximum(h, 0.0)
    out_ref[0, 0, :] = jnp.sum(h * w2_ref[...], axis=1) + b2_ref[0, 0]


def _edge_mlp(ga, gb, ea, wc, b1r, w2r, b2r):
    return pl.pallas_call(
        _mlp_body,
        grid=(N_EBLK,),
        in_specs=[
            pl.BlockSpec((E_BLK, HIDDEN), lambda i: (i, 0)),
            pl.BlockSpec((E_BLK, HIDDEN), lambda i: (i, 0)),
            pl.BlockSpec((E_BLK, EDGE_DIM), lambda i: (i, 0)),
            pl.BlockSpec((EDGE_DIM, HIDDEN), lambda i: (0, 0)),
            pl.BlockSpec((1, HIDDEN), lambda i: (0, 0)),
            pl.BlockSpec((1, HIDDEN), lambda i: (0, 0)),
            pl.BlockSpec(memory_space=pltpu.SMEM),
        ],
        out_specs=pl.BlockSpec((1, 1, E_BLK), lambda i: (i, 0, 0)),
        out_shape=jax.ShapeDtypeStruct((N_EBLK, 1, E_BLK), jnp.float32),
    )(ga, gb, ea, wc, b1r, w2r, b2r)


def kernel(node_emb, edge_index, edge_attr, W1, b1, W2, b2):
    ne = node_emb.astype(jnp.float32)
    src = edge_index[0].astype(jnp.int32).reshape(NW, NCHUNK, CHUNK)
    dst = edge_index[1].astype(jnp.int32).reshape(NW, NCHUNK, CHUNK)
    wa = W1[:HIDDEN]
    wb = W1[HIDDEN:2 * HIDDEN]
    wc = W1[2 * HIDDEN:]
    a, b = _precompute_ab(ne, wa, wb)
    ga, gb = _gather_edges(a, b, src, dst)
    out2d = _edge_mlp(ga, gb, edge_attr, wc,
                      b1.reshape(1, HIDDEN), W2.reshape(1, HIDDEN),
                      b2.reshape(1, 1))
    return out2d.reshape(N_EDGES)


# speedup vs baseline: 3.5711x; 1.2183x over previous
import functools

import jax
import jax.numpy as jnp
from jax import lax
from jax.experimental import pallas as pl
from jax.experimental.pallas import tpu as pltpu
from jax.experimental.pallas import tpu_sc as plsc

N_NODES = 10000
N_EDGES = 320000
HIDDEN = 128
EDGE_DIM = 16

NC = 2
NS = 16
NW = NC * NS
EPW = N_EDGES // NW
CHUNK = 40
NCHUNK = EPW // CHUNK
LANES = 16

E_BLK = 2560
N_EBLK = N_EDGES // E_BLK

NODE_BLK = 1000
N_NBLK = N_NODES // NODE_BLK


def _ab_body(ne_ref, wa_ref, wb_ref, a_ref, b_ref):
    x = ne_ref[...]
    a_ref[...] = jnp.dot(x, wa_ref[...], preferred_element_type=jnp.float32)
    b_ref[...] = jnp.dot(x, wb_ref[...], preferred_element_type=jnp.float32)


def _precompute_ab(ne, wa, wb):
    return pl.pallas_call(
        _ab_body,
        grid=(N_NBLK,),
        in_specs=[
            pl.BlockSpec((NODE_BLK, HIDDEN), lambda i: (i, 0)),
            pl.BlockSpec((HIDDEN, HIDDEN), lambda i: (0, 0)),
            pl.BlockSpec((HIDDEN, HIDDEN), lambda i: (0, 0)),
        ],
        out_specs=[
            pl.BlockSpec((NODE_BLK, HIDDEN), lambda i: (i, 0)),
            pl.BlockSpec((NODE_BLK, HIDDEN), lambda i: (i, 0)),
        ],
        out_shape=[
            jax.ShapeDtypeStruct((N_NODES, HIDDEN), jnp.float32),
            jax.ShapeDtypeStruct((N_NODES, HIDDEN), jnp.float32),
        ],
    )(ne, wa, wb)


def _gather_body(a_hbm, b_hbm, src_hbm, dst_hbm, g_hbm,
                 src_v, dst_v, buf_a0, buf_a1, buf_b0, buf_b1,
                 out0, out1,
                 sga0, sga1, sgb0, sgb1, ssc0, ssc1):
    c = lax.axis_index("c")
    s = lax.axis_index("s")
    wid = s * NC + c
    pltpu.sync_copy(src_hbm.at[wid], src_v)
    pltpu.sync_copy(dst_hbm.at[wid], dst_v)

    buf_a = (buf_a0, buf_a1)
    buf_b = (buf_b0, buf_b1)
    out = (out0, out1)
    sga = (sga0, sga1)
    sgb = (sgb0, sgb1)
    ssc = (ssc0, ssc1)

    def issue_gather(j, k):
        pltpu.async_copy(a_hbm.at[src_v.at[j]], buf_a[k], sga[k])
        pltpu.async_copy(b_hbm.at[dst_v.at[j]], buf_b[k], sgb[k])

    def wait_gather(j, k):
        pltpu.make_async_copy(a_hbm.at[src_v.at[j]], buf_a[k], sga[k]).wait()
        pltpu.make_async_copy(b_hbm.at[dst_v.at[j]], buf_b[k], sgb[k]).wait()

    def issue_scatter(j, k):
        base = wid * EPW + j * CHUNK
        pltpu.async_copy(out[k], g_hbm.at[pl.ds(base, CHUNK)], ssc[k])

    def wait_scatter(j, k):
        base = wid * EPW + j * CHUNK
        pltpu.make_async_copy(out[k], g_hbm.at[pl.ds(base, CHUNK)], ssc[k]).wait()

    issue_gather(0, 0)
    issue_gather(1, 1)

    def body(i, carry):
        for k in (0, 1):
            j = 2 * i + k
            wait_gather(j, k)

            @pl.when(i > 0)
            def _():
                wait_scatter(j - 2, k)

            def add_row(r, cc):
                for v in range(HIDDEN // LANES):
                    sl = pl.ds(v * LANES, LANES)
                    out[k][r, sl] = buf_a[k][r, sl] + buf_b[k][r, sl]
                return cc

            lax.fori_loop(0, CHUNK, add_row, 0)
            issue_scatter(j, k)

            @pl.when(j + 2 < NCHUNK)
            def _():
                issue_gather(j + 2, k)
        return carry

    lax.fori_loop(0, NCHUNK // 2, body, 0)
    wait_scatter(NCHUNK - 2, 0)
    wait_scatter(NCHUNK - 1, 1)


def _gather_edges(a, b, src, dst):
    mesh = plsc.VectorSubcoreMesh(core_axis_name="c", subcore_axis_name="s")
    fn = pl.kernel(
        _gather_body,
        out_type=jax.ShapeDtypeStruct((N_EDGES, HIDDEN), jnp.float32),
        mesh=mesh,
        scratch_types=(
            [pltpu.VMEM((NCHUNK, CHUNK), jnp.int32)] * 2
            + [pltpu.VMEM((CHUNK, HIDDEN), jnp.float32)] * 6
            + [pltpu.SemaphoreType.DMA] * 6
        ),
    )
    return fn(a, b, src, dst)


def _mlp_body(g_ref, ea_ref, wc_ref, b1_ref, w2_ref, b2_ref, out_ref):
    e = jnp.dot(ea_ref[...], wc_ref[...], preferred_element_type=jnp.float32)
    h = g_ref[...] + e + b1_ref[...]
    h = jnp.maximum(h, 0.0)
    out_ref[0, 0, :] = jnp.sum(h * w2_ref[...], axis=1) + b2_ref[0, 0]


def _edge_mlp(g, ea, wc, b1r, w2r, b2r):
    return pl.pallas_call(
        _mlp_body,
        grid=(N_EBLK,),
        in_specs=[
            pl.BlockSpec((E_BLK, HIDDEN), lambda i: (i, 0)),
            pl.BlockSpec((E_BLK, EDGE_DIM), lambda i: (i, 0)),
            pl.BlockSpec((EDGE_DIM, HIDDEN), lambda i: (0, 0)),
            pl.BlockSpec((1, HIDDEN), lambda i: (0, 0)),
            pl.BlockSpec((1, HIDDEN), lambda i: (0, 0)),
            pl.BlockSpec(memory_space=pltpu.SMEM),
        ],
        out_specs=pl.BlockSpec((1, 1, E_BLK), lambda i: (i, 0, 0)),
        out_shape=jax.ShapeDtypeStruct((N_EBLK, 1, E_BLK), jnp.float32),
    )(g, ea, wc, b1r, w2r, b2r)


def kernel(node_emb, edge_index, edge_attr, W1, b1, W2, b2):
    ne = node_emb.astype(jnp.float32)
    src = edge_index[0].astype(jnp.int32).reshape(NW, NCHUNK, CHUNK)
    dst = edge_index[1].astype(jnp.int32).reshape(NW, NCHUNK, CHUNK)
    wa = W1[:HIDDEN]
    wb = W1[HIDDEN:2 * HIDDEN]
    wc = W1[2 * HIDDEN:]
    a, b = _precompute_ab(ne, wa, wb)
    g = _gather_edges(a, b, src, dst)
    out2d = _edge_mlp(g, edge_attr, wc,
                      b1.reshape(1, HIDDEN), W2.reshape(1, HIDDEN),
                      b2.reshape(1, 1))
    return out2d.reshape(N_EDGES)
